# Initial kernel scaffold; baseline (speedup 1.0000x reference)
import jax
import jax.numpy as jnp
from jax.experimental import pallas as pl


def kernel(feature_undist, grouping_indices, point_indices, is_calibrated, ref_rots, rel_rots, points_3d, scales, ref_trans, rel_trans):
    raise NotImplementedError("write your pallas kernel here")


# speedup vs baseline: 5.2129x; 5.2129x over previous
import jax
import jax.numpy as jnp
from jax import lax
from jax.experimental import pallas as pl
from jax.experimental.pallas import tpu as pltpu
from jax.experimental.pallas import tpu_sc as plsc

N = 2_000_000
SUB = 128
NSUB = 5
B = SUB * NSUB
NCHUNK = N // B
L = 16
NC = 2
NW = 32


def _cross(ax, ay, az, bx, by, bz):
    return ay * bz - az * by, az * bx - ax * bz, ax * by - ay * bx


def _rotate_conj(qx, qy, qz, qw, vx, vy, vz):
    tx, ty, tz = _cross(vx, vy, vz, qx, qy, qz)
    tx, ty, tz = 2.0 * tx, 2.0 * ty, 2.0 * tz
    cx, cy, cz = _cross(tx, ty, tz, qx, qy, qz)
    return vx + qw * tx + cx, vy + qw * ty + cy, vz + qw * tz + cz


def _sc_kernel(feat_hbm, gidx_hbm, midx_hbm, pidx_hbm, calw_hbm,
               brx_hbm, bry_hbm, brz_hbm, brw_hbm,
               btx_hbm, bty_hbm, btz_hbm,
               ptx_hbm, pty_hbm, ptz_hbm,
               scales_hbm, relR_hbm, relt_hbm, out_hbm,
               gidx_v, pidx_v, midx_v, scales_v, feat_v, w_v,
               bx_v, by_v, bz_v, bw_v, tx_v, ty_v, tz_v,
               px_v, py_v, pz_v, out_v, relR_v, relt_v,
               ax_v, ay_v, az_v, aw_v, ux_v, uy_v, uz_v,
               sem_lin, sem_gat):
    wid = lax.axis_index("s") * NC + lax.axis_index("c")

    iota = lax.iota(jnp.int32, L)
    c0 = jnp.full((L,), 0, jnp.int32)
    c1 = jnp.full((L,), 1, jnp.int32)
    c2 = jnp.full((L,), 2, jnp.int32)
    c3 = jnp.full((L,), 3, jnp.int32)

    pltpu.sync_copy(relR_hbm, relR_v)
    pltpu.sync_copy(relt_hbm, relt_v)
    rax = plsc.load_gather(relR_v, [iota, c0])
    ray = plsc.load_gather(relR_v, [iota, c1])
    raz = plsc.load_gather(relR_v, [iota, c2])
    raw = plsc.load_gather(relR_v, [iota, c3])
    rtx = plsc.load_gather(relt_v, [iota, c0])
    rty = plsc.load_gather(relt_v, [iota, c1])
    rtz = plsc.load_gather(relt_v, [iota, c2])
    ux, uy, uz = _rotate_conj(rax, ray, raz, raw, rtx, rty, rtz)
    ax_v[...] = rax
    ay_v[...] = ray
    az_v[...] = raz
    aw_v[...] = raw
    ux_v[...] = ux
    uy_v[...] = uy
    uz_v[...] = uz

    def group_body(g, carry):
        b16 = g * L
        lids = b16 + iota
        d16 = pl.ds(b16, L)
        m = midx_v[d16]
        s = scales_v[d16]
        wgt = 0.5 + 0.5 * w_v[d16]
        bx, by, bz, bw = bx_v[d16], by_v[d16], bz_v[d16], bw_v[d16]
        tx, ty, tz = tx_v[d16], ty_v[d16], tz_v[d16]
        px, py, pz = px_v[d16], py_v[d16], pz_v[d16]
        fx = plsc.load_gather(feat_v, [lids, c0])
        fy = plsc.load_gather(feat_v, [lids, c1])
        fz = plsc.load_gather(feat_v, [lids, c2])
        ax = plsc.load_gather(ax_v, [m])
        ay = plsc.load_gather(ay_v, [m])
        az = plsc.load_gather(az_v, [m])
        aw = plsc.load_gather(aw_v, [m])
        mux = plsc.load_gather(ux_v, [m])
        muy = plsc.load_gather(uy_v, [m])
        muz = plsc.load_gather(uz_v, [m])

        qw = aw * bw - ax * bx - ay * by - az * bz
        qx = aw * bx + ax * bw + ay * bz - az * by
        qy = aw * by - ax * bz + ay * bw + az * bx
        qz = aw * bz + ax * by - ay * bx + az * bw

        vx, vy, vz = mux + tx, muy + ty, muz + tz
        r1x, r1y, r1z = _rotate_conj(bx, by, bz, bw, vx, vy, vz)
        r2x, r2y, r2z = _rotate_conj(qx, qy, qz, qw, fx, fy, fz)

        ox = wgt * (px + r1x - s * r2x)
        oy = wgt * (py + r1y - s * r2y)
        oz = wgt * (pz + r1z - s * r2z)
        plsc.store_scatter(out_v, [lids, c0], ox)
        plsc.store_scatter(out_v, [lids, c1], oy)
        plsc.store_scatter(out_v, [lids, c2], oz)
        return carry

    def chunk_body(i, carry):
        c = wid + i * NW
        base = c * B
        hs = [
            pltpu.async_copy(feat_hbm.at[pl.ds(base, B)], feat_v, sem_lin),
            pltpu.async_copy(midx_hbm.at[pl.ds(base, B)], midx_v, sem_lin),
            pltpu.async_copy(scales_hbm.at[pl.ds(base, B)], scales_v, sem_lin),
        ]
        for j in range(NSUB):
            hs.append(pltpu.async_copy(
                gidx_hbm.at[pl.ds(base + j * SUB, SUB)], gidx_v.at[j], sem_lin))
            hs.append(pltpu.async_copy(
                pidx_hbm.at[pl.ds(base + j * SUB, SUB)], pidx_v.at[j], sem_lin))
        for h in hs:
            h.wait()
        gs = []
        for j in range(NSUB):
            d = pl.ds(j * SUB, SUB)
            gidx_j = gidx_v.at[j]
            pidx_j = pidx_v.at[j]
            gs += [
                pltpu.async_copy(calw_hbm.at[gidx_j], w_v.at[d], sem_gat),
                pltpu.async_copy(brx_hbm.at[gidx_j], bx_v.at[d], sem_gat),
                pltpu.async_copy(bry_hbm.at[gidx_j], by_v.at[d], sem_gat),
                pltpu.async_copy(brz_hbm.at[gidx_j], bz_v.at[d], sem_gat),
                pltpu.async_copy(brw_hbm.at[gidx_j], bw_v.at[d], sem_gat),
                pltpu.async_copy(btx_hbm.at[gidx_j], tx_v.at[d], sem_gat),
                pltpu.async_copy(bty_hbm.at[gidx_j], ty_v.at[d], sem_gat),
                pltpu.async_copy(btz_hbm.at[gidx_j], tz_v.at[d], sem_gat),
                pltpu.async_copy(ptx_hbm.at[pidx_j], px_v.at[d], sem_gat),
                pltpu.async_copy(pty_hbm.at[pidx_j], py_v.at[d], sem_gat),
                pltpu.async_copy(ptz_hbm.at[pidx_j], pz_v.at[d], sem_gat),
            ]
        for h in gs:
            h.wait()
        lax.fori_loop(0, B // L, group_body, 0)
        pltpu.sync_copy(out_v, out_hbm.at[pl.ds(base, B)])
        return carry

    niter = (NCHUNK - wid + NW - 1) // NW
    lax.fori_loop(0, niter, chunk_body, 0)


def kernel(feature_undist, grouping_indices, point_indices, is_calibrated,
           ref_rots, rel_rots, points_3d, scales, ref_trans, rel_trans):
    gidx = grouping_indices[:, 0]
    midx = grouping_indices[:, 1]
    calw = is_calibrated.astype(jnp.float32)
    scales_flat = scales.reshape(N)

    mesh = plsc.VectorSubcoreMesh(core_axis_name="c", subcore_axis_name="s")
    f32, i32 = jnp.float32, jnp.int32
    run = pl.kernel(
        _sc_kernel, mesh=mesh,
        out_type=jax.ShapeDtypeStruct((N, 3), f32),
        compiler_params=pltpu.CompilerParams(
            needs_layout_passes=False, use_tc_tiling_on_sc=False),
        scratch_types=[
            pltpu.VMEM((NSUB, SUB), i32),
            pltpu.VMEM((NSUB, SUB), i32),
            pltpu.VMEM((B,), i32),
            pltpu.VMEM((B,), f32),
            pltpu.VMEM((B, 3), f32),
            pltpu.VMEM((B,), f32),
            pltpu.VMEM((B,), f32),
            pltpu.VMEM((B,), f32),
            pltpu.VMEM((B,), f32),
            pltpu.VMEM((B,), f32),
            pltpu.VMEM((B,), f32),
            pltpu.VMEM((B,), f32),
            pltpu.VMEM((B,), f32),
            pltpu.VMEM((B,), f32),
            pltpu.VMEM((B,), f32),
            pltpu.VMEM((B,), f32),
            pltpu.VMEM((B, 3), f32),
            pltpu.VMEM((16, 4), f32),
            pltpu.VMEM((16, 3), f32),
            pltpu.VMEM((16,), f32),
            pltpu.VMEM((16,), f32),
            pltpu.VMEM((16,), f32),
            pltpu.VMEM((16,), f32),
            pltpu.VMEM((16,), f32),
            pltpu.VMEM((16,), f32),
            pltpu.VMEM((16,), f32),
            pltpu.SemaphoreType.DMA,
            pltpu.SemaphoreType.DMA,
        ],
    )
    return run(feature_undist, gidx, midx, point_indices, calw,
               ref_rots[:, 0], ref_rots[:, 1], ref_rots[:, 2], ref_rots[:, 3],
               ref_trans[:, 0], ref_trans[:, 1], ref_trans[:, 2],
               points_3d[:, 0], points_3d[:, 1], points_3d[:, 2],
               scales_flat, rel_rots, rel_trans)


# speedup vs baseline: 31.5781x; 6.0577x over previous
import jax
import jax.numpy as jnp
from jax import lax
from jax.experimental import pallas as pl
from jax.experimental.pallas import tpu as pltpu
from jax.experimental.pallas import tpu_sc as plsc

N = 2_000_000
SUB = 128
NSUB = 5
B = SUB * NSUB
NCHUNK = N // B
L = 16
NC = 2
NW = 32


def _cross(ax, ay, az, bx, by, bz):
    return ay * bz - az * by, az * bx - ax * bz, ax * by - ay * bx


def _rotate_conj(qx, qy, qz, qw, vx, vy, vz):
    tx, ty, tz = _cross(vx, vy, vz, qx, qy, qz)
    tx, ty, tz = 2.0 * tx, 2.0 * ty, 2.0 * tz
    cx, cy, cz = _cross(tx, ty, tz, qx, qy, qz)
    return vx + qw * tx + cx, vy + qw * ty + cy, vz + qw * tz + cz


def _sc_kernel(fx_hbm, fy_hbm, fz_hbm, gidx_hbm, midx_hbm, pidx_hbm, calw_hbm,
               brx_hbm, bry_hbm, brz_hbm, brw_hbm,
               btx_hbm, bty_hbm, btz_hbm,
               ptx_hbm, pty_hbm, ptz_hbm,
               scales_hbm, relf_hbm,
               ox_hbm, oy_hbm, oz_hbm,
               gidx_v, pidx_v, midx_v, scales_v,
               fx_v, fy_v, fz_v, w_v,
               bx_v, by_v, bz_v, bw_v, tx_v, ty_v, tz_v,
               px_v, py_v, pz_v, ox_v, oy_v, oz_v, relf_v,
               ax_v, ay_v, az_v, aw_v, ux_v, uy_v, uz_v,
               sem_lin, sem_gat):
    wid = lax.axis_index("s") * NC + lax.axis_index("c")

    iota = lax.iota(jnp.int32, L)

    pltpu.sync_copy(relf_hbm, relf_v)
    i4 = iota * 4
    rax = plsc.load_gather(relf_v, [i4])
    ray = plsc.load_gather(relf_v, [i4 + 1])
    raz = plsc.load_gather(relf_v, [i4 + 2])
    raw = plsc.load_gather(relf_v, [i4 + 3])
    i3 = iota * 3 + 64
    rtx = plsc.load_gather(relf_v, [i3])
    rty = plsc.load_gather(relf_v, [i3 + 1])
    rtz = plsc.load_gather(relf_v, [i3 + 2])
    ux, uy, uz = _rotate_conj(rax, ray, raz, raw, rtx, rty, rtz)
    ax_v[...] = rax
    ay_v[...] = ray
    az_v[...] = raz
    aw_v[...] = raw
    ux_v[...] = ux
    uy_v[...] = uy
    uz_v[...] = uz

    def group_body(g, carry):
        b16 = g * L
        d16 = pl.ds(b16, L)
        m = midx_v[d16]
        s = scales_v[d16]
        wgt = 0.5 + 0.5 * w_v[d16]
        bx, by, bz, bw = bx_v[d16], by_v[d16], bz_v[d16], bw_v[d16]
        tx, ty, tz = tx_v[d16], ty_v[d16], tz_v[d16]
        px, py, pz = px_v[d16], py_v[d16], pz_v[d16]
        fx, fy, fz = fx_v[d16], fy_v[d16], fz_v[d16]
        ax = plsc.load_gather(ax_v, [m])
        ay = plsc.load_gather(ay_v, [m])
        az = plsc.load_gather(az_v, [m])
        aw = plsc.load_gather(aw_v, [m])
        mux = plsc.load_gather(ux_v, [m])
        muy = plsc.load_gather(uy_v, [m])
        muz = plsc.load_gather(uz_v, [m])

        qw = aw * bw - ax * bx - ay * by - az * bz
        qx = aw * bx + ax * bw + ay * bz - az * by
        qy = aw * by - ax * bz + ay * bw + az * bx
        qz = aw * bz + ax * by - ay * bx + az * bw

        vx, vy, vz = mux + tx, muy + ty, muz + tz
        r1x, r1y, r1z = _rotate_conj(bx, by, bz, bw, vx, vy, vz)
        r2x, r2y, r2z = _rotate_conj(qx, qy, qz, qw, fx, fy, fz)

        ox_v[d16] = wgt * (px + r1x - s * r2x)
        oy_v[d16] = wgt * (py + r1y - s * r2y)
        oz_v[d16] = wgt * (pz + r1z - s * r2z)
        return carry

    def chunk_body(i, carry):
        c = wid + i * NW
        base = c * B
        dB = pl.ds(base, B)
        hs = [
            pltpu.async_copy(fx_hbm.at[dB], fx_v, sem_lin),
            pltpu.async_copy(fy_hbm.at[dB], fy_v, sem_lin),
            pltpu.async_copy(fz_hbm.at[dB], fz_v, sem_lin),
            pltpu.async_copy(midx_hbm.at[dB], midx_v, sem_lin),
            pltpu.async_copy(scales_hbm.at[dB], scales_v, sem_lin),
        ]
        for j in range(NSUB):
            hs.append(pltpu.async_copy(
                gidx_hbm.at[pl.ds(base + j * SUB, SUB)], gidx_v.at[j], sem_lin))
            hs.append(pltpu.async_copy(
                pidx_hbm.at[pl.ds(base + j * SUB, SUB)], pidx_v.at[j], sem_lin))
        for h in hs:
            h.wait()
        gs = []
        for j in range(NSUB):
            d = pl.ds(j * SUB, SUB)
            gidx_j = gidx_v.at[j]
            pidx_j = pidx_v.at[j]
            gs += [
                pltpu.async_copy(calw_hbm.at[gidx_j], w_v.at[d], sem_gat),
                pltpu.async_copy(brx_hbm.at[gidx_j], bx_v.at[d], sem_gat),
                pltpu.async_copy(bry_hbm.at[gidx_j], by_v.at[d], sem_gat),
                pltpu.async_copy(brz_hbm.at[gidx_j], bz_v.at[d], sem_gat),
                pltpu.async_copy(brw_hbm.at[gidx_j], bw_v.at[d], sem_gat),
                pltpu.async_copy(btx_hbm.at[gidx_j], tx_v.at[d], sem_gat),
                pltpu.async_copy(bty_hbm.at[gidx_j], ty_v.at[d], sem_gat),
                pltpu.async_copy(btz_hbm.at[gidx_j], tz_v.at[d], sem_gat),
                pltpu.async_copy(ptx_hbm.at[pidx_j], px_v.at[d], sem_gat),
                pltpu.async_copy(pty_hbm.at[pidx_j], py_v.at[d], sem_gat),
                pltpu.async_copy(ptz_hbm.at[pidx_j], pz_v.at[d], sem_gat),
            ]
        for h in gs:
            h.wait()
        lax.fori_loop(0, B // L, group_body, 0)
        os = [
            pltpu.async_copy(ox_v, ox_hbm.at[dB], sem_lin),
            pltpu.async_copy(oy_v, oy_hbm.at[dB], sem_lin),
            pltpu.async_copy(oz_v, oz_hbm.at[dB], sem_lin),
        ]
        for h in os:
            h.wait()
        return carry

    niter = (NCHUNK - wid + NW - 1) // NW
    lax.fori_loop(0, niter, chunk_body, 0)


def kernel(feature_undist, grouping_indices, point_indices, is_calibrated,
           ref_rots, rel_rots, points_3d, scales, ref_trans, rel_trans):
    gidx = grouping_indices[:, 0]
    midx = grouping_indices[:, 1]
    calw = is_calibrated.astype(jnp.float32)
    scales_flat = scales.reshape(N)
    relf = jnp.concatenate([rel_rots.reshape(64), rel_trans.reshape(48)])

    mesh = plsc.VectorSubcoreMesh(core_axis_name="c", subcore_axis_name="s")
    f32, i32 = jnp.float32, jnp.int32
    run = pl.kernel(
        _sc_kernel, mesh=mesh,
        out_type=(jax.ShapeDtypeStruct((N,), f32),) * 3,
        compiler_params=pltpu.CompilerParams(
            needs_layout_passes=False, use_tc_tiling_on_sc=False),
        scratch_types=[
            pltpu.VMEM((NSUB, SUB), i32),
            pltpu.VMEM((NSUB, SUB), i32),
            pltpu.VMEM((B,), i32),
            pltpu.VMEM((B,), f32),
            pltpu.VMEM((B,), f32),
            pltpu.VMEM((B,), f32),
            pltpu.VMEM((B,), f32),
            pltpu.VMEM((B,), f32),
            pltpu.VMEM((B,), f32),
            pltpu.VMEM((B,), f32),
            pltpu.VMEM((B,), f32),
            pltpu.VMEM((B,), f32),
            pltpu.VMEM((B,), f32),
            pltpu.VMEM((B,), f32),
            pltpu.VMEM((B,), f32),
            pltpu.VMEM((B,), f32),
            pltpu.VMEM((B,), f32),
            pltpu.VMEM((B,), f32),
            pltpu.VMEM((B,), f32),
            pltpu.VMEM((B,), f32),
            pltpu.VMEM((B,), f32),
            pltpu.VMEM((112,), f32),
            pltpu.VMEM((16,), f32),
            pltpu.VMEM((16,), f32),
            pltpu.VMEM((16,), f32),
            pltpu.VMEM((16,), f32),
            pltpu.VMEM((16,), f32),
            pltpu.VMEM((16,), f32),
            pltpu.VMEM((16,), f32),
            pltpu.SemaphoreType.DMA,
            pltpu.SemaphoreType.DMA,
        ],
    )
    ox, oy, oz = run(
        feature_undist[:, 0], feature_undist[:, 1], feature_undist[:, 2],
        gidx, midx, point_indices, calw,
        ref_rots[:, 0], ref_rots[:, 1], ref_rots[:, 2], ref_rots[:, 3],
        ref_trans[:, 0], ref_trans[:, 1], ref_trans[:, 2],
        points_3d[:, 0], points_3d[:, 1], points_3d[:, 2],
        scales_flat, relf)
    return jnp.stack([ox, oy, oz], axis=1)
